# baseline (speedup 1.0000x reference)
import functools

import jax
import jax.numpy as jnp
from jax import lax
from jax.experimental import pallas as pl
from jax.experimental.pallas import tpu as pltpu
from jax.experimental.pallas import tpu_sc as plsc

N = 4096
D = 768
E = 16
I_DIM = 3072
CAP = 320
SLOTS = E * CAP
DUMMY = SLOTS
KSPLIT = 1
IC = I_DIM // KSPLIT
CHUNK = 512
SC_CORES = 2
SC_SUBCORES = 16
SC_W = SC_CORES * SC_SUBCORES
TPW = N // SC_W


_QMAX = 1 << 25
_QBASE = 0x3D800000


def _excl_cumsum_tokens(a):
    s = a
    k = 1
    while k < N:
        shifted = jnp.concatenate(
            [jnp.zeros((k, E), jnp.int32), s[:N - k]], axis=0)
        s = s + shifted
        k *= 2
    return s - a


def _route_select_body(xf_ref, sw_ref, sb_ref,
                       p_ref, dst_ref, pos_ref, kept_ref):
    xf = xf_ref[...]
    sw = sw_ref[...]
    logits = lax.dot_general(
        xf.astype(jnp.bfloat16), sw.astype(jnp.bfloat16),
        (((1,), (1,)), ((), ())),
        preferred_element_type=jnp.float32,
    ) + sb_ref[...]
    m = jnp.max(logits, axis=1, keepdims=True)
    ex = jnp.exp(logits - m)
    s = jnp.sum(ex, axis=1, keepdims=True)
    probs = ex / s
    pm = jnp.max(probs, axis=1, keepdims=True)
    cols = lax.broadcasted_iota(jnp.int32, (N, E), 1)
    route = jnp.min(jnp.where(probs == pm, cols, E), axis=1, keepdims=True)
    onehot = cols == route

    prio = lax.bitcast_convert_type(pm, jnp.int32)
    q = jnp.clip(prio - _QBASE, 0, _QMAX)

    def bis(_, carry):
        lo, hi = carry
        mid = (lo + hi) >> 1
        ge = jnp.sum((onehot & (q >= mid)).astype(jnp.int32),
                     axis=0, keepdims=True)
        take = ge >= CAP
        return jnp.where(take, mid, lo), jnp.where(take, hi, mid)

    lo0 = jnp.zeros((1, E), jnp.int32)
    hi0 = jnp.full((1, E), _QMAX + 1, jnp.int32)
    tau, _ = lax.fori_loop(0, 26, bis, (lo0, hi0))

    above = onehot & (q > tau)
    above_cnt = jnp.sum(above.astype(jnp.int32), axis=0, keepdims=True)
    room = CAP - above_cnt
    at = onehot & (q == tau)
    tie_rank = _excl_cumsum_tokens(at.astype(jnp.int32))
    keep_oh = above | (at & (tie_rank < room))
    slot_oh = _excl_cumsum_tokens(keep_oh.astype(jnp.int32))
    kept = jnp.sum(keep_oh.astype(jnp.int32), axis=1, keepdims=True)
    slot = jnp.sum(jnp.where(keep_oh, slot_oh, 0), axis=1, keepdims=True)
    pos = route * CAP + slot
    p_ref[...] = pm
    dst_ref[...] = jnp.where(kept != 0, pos, DUMMY)
    pos_ref[...] = jnp.where(kept != 0, pos, 0)
    kept_ref[...] = kept


IC2 = I_DIM // 2
DH = D // 2


def _ffn_body(xg_ref, w1_ref, b1_ref, w2_ref, b2_ref, yg_ref, h_ref):
    k = pl.program_id(1)

    @pl.when(k < 2)
    def _():
        xb = xg_ref[...].astype(jnp.bfloat16)
        w1 = w1_ref[0].astype(jnp.bfloat16)
        h = lax.dot_general(xb, w1, (((1,), (1,)), ((), ())),
                            preferred_element_type=jnp.float32)
        h = h + b1_ref[0]
        h = 0.5 * h * (1.0 + lax.erf(h * 0.7071067811865476))
        h_ref[:, pl.ds(pl.multiple_of(k * IC2, IC2), IC2)] = h

    @pl.when(k >= 2)
    def _():
        w2 = w2_ref[0].astype(jnp.bfloat16)
        hb = h_ref[...].astype(jnp.bfloat16)
        y = lax.dot_general(hb, w2, (((1,), (1,)), ((), ())),
                            preferred_element_type=jnp.float32)
        yg_ref[...] = y + b2_ref[0]


def _combine_body(yt_ref, xf_ref, kept_ref, p_ref, out_ref):
    keep = kept_ref[...] != 0
    val = jnp.where(keep, yt_ref[...], xf_ref[...])
    out_ref[...] = val * p_ref[...]


@functools.cache
def _sc_kernels():
    mesh = plsc.VectorSubcoreMesh(core_axis_name="c", subcore_axis_name="s",
                                  num_cores=SC_CORES,
                                  num_subcores=SC_SUBCORES)

    @functools.partial(
        pl.kernel, mesh=mesh,
        out_type=jax.ShapeDtypeStruct((SLOTS + 1, D), jnp.float32),
        scratch_types=[pltpu.VMEM((TPW,), jnp.int32),
                       pltpu.VMEM((TPW, D), jnp.float32),
                       pltpu.SemaphoreType.DMA,
                       pltpu.SemaphoreType.DMA],
    )
    def sc_scatter(x_hbm, dst_hbm, xg_hbm, idx_v, rows_v, sem, sem2):
        wid = lax.axis_index("s") * SC_CORES + lax.axis_index("c")
        base = wid * TPW
        c1 = pltpu.async_copy(dst_hbm.at[pl.ds(base, TPW)], idx_v, sem)
        c2 = pltpu.async_copy(x_hbm.at[pl.ds(base, TPW)], rows_v, sem2)
        c1.wait()
        c2.wait()
        pltpu.async_copy(rows_v, xg_hbm.at[idx_v], sem).wait()

    @functools.partial(
        pl.kernel, mesh=mesh,
        out_type=jax.ShapeDtypeStruct((N, D), jnp.float32),
        scratch_types=[pltpu.VMEM((TPW,), jnp.int32),
                       pltpu.VMEM((TPW, D), jnp.float32),
                       pltpu.SemaphoreType.DMA],
    )
    def sc_gather(yg_hbm, pos_hbm, yt_hbm, idx_v, rows_v, sem):
        wid = lax.axis_index("s") * SC_CORES + lax.axis_index("c")
        base = wid * TPW
        pltpu.async_copy(pos_hbm.at[pl.ds(base, TPW)], idx_v, sem).wait()
        pltpu.async_copy(yg_hbm.at[idx_v], rows_v, sem).wait()
        pltpu.async_copy(rows_v, yt_hbm.at[pl.ds(base, TPW)], sem).wait()

    return sc_scatter, sc_gather


def kernel(x, switch_W, switch_b, w1, b1, w2, b2):
    xf = x.reshape(N, D)
    p_col, dst_col, pos_col, kept_col = pl.pallas_call(
        _route_select_body,
        out_shape=[jax.ShapeDtypeStruct((N, 1), jnp.float32),
                   jax.ShapeDtypeStruct((N, 1), jnp.int32),
                   jax.ShapeDtypeStruct((N, 1), jnp.int32),
                   jax.ShapeDtypeStruct((N, 1), jnp.int32)],
    )(xf, switch_W, switch_b.reshape(1, E))

    sc_scatter, sc_gather = _sc_kernels()
    xg = sc_scatter(xf, dst_col.reshape(N))

    yg = pl.pallas_call(
        _ffn_body,
        grid=(E, 4),
        in_specs=[
            pl.BlockSpec((CAP, D), lambda e, k: (e, 0)),
            pl.BlockSpec((1, IC2, D), lambda e, k: (e, jnp.minimum(k, 1), 0)),
            pl.BlockSpec((1, 1, IC2), lambda e, k: (e, 0, jnp.minimum(k, 1))),
            pl.BlockSpec((1, DH, I_DIM),
                         lambda e, k: (e, jnp.maximum(k - 2, 0), 0)),
            pl.BlockSpec((1, 1, DH),
                         lambda e, k: (e, 0, jnp.maximum(k - 2, 0))),
        ],
        out_specs=pl.BlockSpec((CAP, DH), lambda e, k: (e, jnp.maximum(k - 2, 0))),
        out_shape=jax.ShapeDtypeStruct((SLOTS, D), jnp.float32),
        scratch_shapes=[pltpu.VMEM((CAP, I_DIM), jnp.float32)],
    )(xg, w1, b1.reshape(E, 1, I_DIM), w2, b2.reshape(E, 1, D))

    yt = sc_gather(yg, pos_col.reshape(N))

    out = pl.pallas_call(
        _combine_body,
        grid=(N // CHUNK,),
        in_specs=[pl.BlockSpec((CHUNK, D), lambda c: (c, 0)),
                  pl.BlockSpec((CHUNK, D), lambda c: (c, 0)),
                  pl.BlockSpec((CHUNK, 1), lambda c: (c, 0)),
                  pl.BlockSpec((CHUNK, 1), lambda c: (c, 0))],
        out_specs=pl.BlockSpec((CHUNK, D), lambda c: (c, 0)),
        out_shape=jax.ShapeDtypeStruct((N, D), jnp.float32),
    )(yt, xf, kept_col, p_col)

    return out.reshape(x.shape)


# speedup vs baseline: 1.1970x; 1.1970x over previous
import functools

import jax
import jax.numpy as jnp
from jax import lax
from jax.experimental import pallas as pl
from jax.experimental.pallas import tpu as pltpu
from jax.experimental.pallas import tpu_sc as plsc

N = 4096
D = 768
E = 16
I_DIM = 3072
CAP = 320
SLOTS = E * CAP
DUMMY = SLOTS
KSPLIT = 1
IC = I_DIM // KSPLIT
CHUNK = 512
SC_CORES = 2
SC_SUBCORES = 16
SC_W = SC_CORES * SC_SUBCORES
TPW = N // SC_W


_QMAX = 1 << 25
_QBASE = 0x3D800000


def _excl_cumsum_tokens(a):
    s = a
    k = 1
    while k < N:
        shifted = jnp.concatenate(
            [jnp.zeros((k, E), jnp.int32), s[:N - k]], axis=0)
        s = s + shifted
        k *= 2
    return s - a


def _route_select_body(xf_ref, sw_ref, sb_ref,
                       p_ref, dst_ref, pos_ref, kept_ref):
    xf = xf_ref[...]
    sw = sw_ref[...]
    logits = lax.dot_general(
        xf.astype(jnp.bfloat16), sw.astype(jnp.bfloat16),
        (((1,), (1,)), ((), ())),
        preferred_element_type=jnp.float32,
    ) + sb_ref[...]
    m = jnp.max(logits, axis=1, keepdims=True)
    ex = jnp.exp(logits - m)
    s = jnp.sum(ex, axis=1, keepdims=True)
    probs = ex / s
    pm = jnp.max(probs, axis=1, keepdims=True)
    cols = lax.broadcasted_iota(jnp.int32, (N, E), 1)
    route = jnp.min(jnp.where(probs == pm, cols, E), axis=1, keepdims=True)
    onehot = cols == route

    prio = lax.bitcast_convert_type(pm, jnp.int32)
    q = jnp.clip(prio - _QBASE, 0, _QMAX)

    def bis(_, carry):
        lo, hi = carry
        mid = (lo + hi) >> 1
        ge = jnp.sum((onehot & (q >= mid)).astype(jnp.int32),
                     axis=0, keepdims=True)
        take = ge >= CAP
        return jnp.where(take, mid, lo), jnp.where(take, hi, mid)

    lo0 = jnp.zeros((1, E), jnp.int32)
    hi0 = jnp.full((1, E), _QMAX + 1, jnp.int32)
    tau, _ = lax.fori_loop(0, 26, bis, (lo0, hi0))

    above = onehot & (q > tau)
    above_cnt = jnp.sum(above.astype(jnp.int32), axis=0, keepdims=True)
    room = CAP - above_cnt
    at = onehot & (q == tau)
    tie_rank = _excl_cumsum_tokens(at.astype(jnp.int32))
    keep_oh = above | (at & (tie_rank < room))
    slot_oh = _excl_cumsum_tokens(keep_oh.astype(jnp.int32))
    kept = jnp.sum(keep_oh.astype(jnp.int32), axis=1, keepdims=True)
    slot = jnp.sum(jnp.where(keep_oh, slot_oh, 0), axis=1, keepdims=True)
    pos = route * CAP + slot
    p_ref[...] = pm
    dst_ref[...] = jnp.where(kept != 0, pos, DUMMY)
    pos_ref[...] = jnp.where(kept != 0, pos, 0)
    kept_ref[...] = kept


def _ffn_body(xg_ref, w1_ref, b1_ref, w2_ref, b2_ref, yg_ref):
    xb = xg_ref[...].astype(jnp.bfloat16)
    w1 = w1_ref[0].astype(jnp.bfloat16)
    h = lax.dot_general(xb, w1, (((1,), (1,)), ((), ())),
                        preferred_element_type=jnp.float32)
    h = h + b1_ref[0]
    h = 0.5 * h * (1.0 + lax.erf(h * 0.7071067811865476))
    w2 = w2_ref[0].astype(jnp.bfloat16)
    y = lax.dot_general(h.astype(jnp.bfloat16), w2, (((1,), (1,)), ((), ())),
                        preferred_element_type=jnp.float32)
    yg_ref[...] = y + b2_ref[0]


def _combine_body(yt_ref, xf_ref, kept_ref, p_ref, out_ref):
    keep = kept_ref[...] != 0
    val = jnp.where(keep, yt_ref[...], xf_ref[...])
    out_ref[...] = val * p_ref[...]


@functools.cache
def _sc_kernels():
    mesh = plsc.VectorSubcoreMesh(core_axis_name="c", subcore_axis_name="s",
                                  num_cores=SC_CORES,
                                  num_subcores=SC_SUBCORES)

    @functools.partial(
        pl.kernel, mesh=mesh,
        out_type=jax.ShapeDtypeStruct((SLOTS + 1, D), jnp.float32),
        scratch_types=[pltpu.VMEM((TPW,), jnp.int32),
                       pltpu.VMEM((TPW, D), jnp.float32),
                       pltpu.SemaphoreType.DMA],
    )
    def sc_scatter(x_hbm, dst_hbm, xg_hbm, idx_v, rows_v, sem):
        wid = lax.axis_index("s") * SC_CORES + lax.axis_index("c")
        base = wid * TPW
        pltpu.async_copy(dst_hbm.at[pl.ds(base, TPW)], idx_v, sem).wait()
        pltpu.async_copy(x_hbm.at[pl.ds(base, TPW)], rows_v, sem).wait()
        pltpu.async_copy(rows_v, xg_hbm.at[idx_v], sem).wait()

    @functools.partial(
        pl.kernel, mesh=mesh,
        out_type=jax.ShapeDtypeStruct((N, D), jnp.float32),
        scratch_types=[pltpu.VMEM((TPW,), jnp.int32),
                       pltpu.VMEM((TPW, D), jnp.float32),
                       pltpu.SemaphoreType.DMA],
    )
    def sc_gather(yg_hbm, pos_hbm, yt_hbm, idx_v, rows_v, sem):
        wid = lax.axis_index("s") * SC_CORES + lax.axis_index("c")
        base = wid * TPW
        pltpu.async_copy(pos_hbm.at[pl.ds(base, TPW)], idx_v, sem).wait()
        pltpu.async_copy(yg_hbm.at[idx_v], rows_v, sem).wait()
        pltpu.async_copy(rows_v, yt_hbm.at[pl.ds(base, TPW)], sem).wait()

    return sc_scatter, sc_gather


def kernel(x, switch_W, switch_b, w1, b1, w2, b2):
    xf = x.reshape(N, D)
    p_col, dst_col, pos_col, kept_col = pl.pallas_call(
        _route_select_body,
        out_shape=[jax.ShapeDtypeStruct((N, 1), jnp.float32),
                   jax.ShapeDtypeStruct((N, 1), jnp.int32),
                   jax.ShapeDtypeStruct((N, 1), jnp.int32),
                   jax.ShapeDtypeStruct((N, 1), jnp.int32)],
    )(xf, switch_W, switch_b.reshape(1, E))

    sc_scatter, sc_gather = _sc_kernels()
    xg = sc_scatter(xf, dst_col.reshape(N))

    yg = pl.pallas_call(
        _ffn_body,
        grid=(E,),
        in_specs=[pl.BlockSpec((CAP, D), lambda e: (e, 0)),
                  pl.BlockSpec((1, IC, D), lambda e: (e, 0, 0)),
                  pl.BlockSpec((1, 1, IC), lambda e: (e, 0, 0)),
                  pl.BlockSpec((1, D, IC), lambda e: (e, 0, 0)),
                  pl.BlockSpec((1, 1, D), lambda e: (e, 0, 0))],
        out_specs=pl.BlockSpec((CAP, D), lambda e: (e, 0)),
        out_shape=jax.ShapeDtypeStruct((SLOTS, D), jnp.float32),
    )(xg, w1, b1.reshape(E, 1, I_DIM), w2, b2.reshape(E, 1, D))

    yt = sc_gather(yg, pos_col.reshape(N))

    out = pl.pallas_call(
        _combine_body,
        grid=(N // CHUNK,),
        in_specs=[pl.BlockSpec((CHUNK, D), lambda c: (c, 0)),
                  pl.BlockSpec((CHUNK, D), lambda c: (c, 0)),
                  pl.BlockSpec((CHUNK, 1), lambda c: (c, 0)),
                  pl.BlockSpec((CHUNK, 1), lambda c: (c, 0))],
        out_specs=pl.BlockSpec((CHUNK, D), lambda c: (c, 0)),
        out_shape=jax.ShapeDtypeStruct((N, D), jnp.float32),
    )(yt, xf, kept_col, p_col)

    return out.reshape(x.shape)
